# baseline (speedup 1.0000x reference)
import functools

import jax
import jax.numpy as jnp
from jax import lax
from jax.experimental import pallas as pl
from jax.experimental.pallas import tpu as pltpu
from jax.experimental.pallas import tpu_sc as plsc

NC = 2
NS = 16
ECH = 128
SB = 40


def _round_up(a, b):
    return (a + b - 1) // b * b


def _sc_mesh():
    return plsc.VectorSubcoreMesh(
        core_axis_name="c", subcore_axis_name="s", num_cores=NC,
        num_subcores=NS)


def _make_deg_kernel(n_pad, n_chunks):
    zrows = n_pad // NS

    @functools.partial(
        pl.kernel,
        out_type=jax.ShapeDtypeStruct((n_pad,), jnp.float32),
        mesh=_sc_mesh(),
        scratch_types=[
            pltpu.VMEM((n_chunks, ECH), jnp.int32),
            pltpu.VMEM((ECH,), jnp.float32),
            pltpu.VMEM((zrows,), jnp.float32),
            pltpu.VMEM_SHARED((n_pad,), jnp.float32),
        ],
    )
    def deg_kernel(col_hbm, deg_hbm, idx_v, ones_v, zero_v, acc):
        cid = lax.axis_index("c")
        sid = lax.axis_index("s")

        @pl.when(cid == 0)
        def _():
            for k in range(ECH // 16):
                ones_v[pl.ds(16 * k, 16)] = jnp.ones((16,), jnp.float32)

            @pl.loop(0, zrows // 16)
            def _(i):
                zero_v[pl.ds(i * 16, 16)] = jnp.zeros((16,), jnp.float32)

            pltpu.sync_copy(zero_v, acc.at[pl.ds(sid * zrows, zrows)])
            pltpu.sync_copy(col_hbm.at[sid], idx_v)
            plsc.subcore_barrier()

            @pl.loop(0, n_chunks)
            def _(j):
                pltpu.sync_copy(ones_v, acc.at[idx_v.at[j]], add=True)

            plsc.subcore_barrier()
            pltpu.sync_copy(acc.at[pl.ds(sid * zrows, zrows)],
                            deg_hbm.at[pl.ds(sid * zrows, zrows)])

    return deg_kernel


def _make_agg_kernel(n, n_pad, dh, n_chunks):
    zrows = n_pad // NS
    orows = _round_up(-(-n // NS), 8)
    last = n - orows * (NS - 1)
    nb = n_chunks // SB

    @functools.partial(
        pl.kernel,
        out_type=(jax.ShapeDtypeStruct((n, dh), jnp.float32),
                  jax.ShapeDtypeStruct((n, dh), jnp.float32)),
        mesh=_sc_mesh(),
        scratch_types=[
            pltpu.VMEM((SB, ECH), jnp.int32),
            pltpu.VMEM((SB, ECH), jnp.int32),
            pltpu.VMEM((ECH, dh), jnp.float32),
            pltpu.VMEM((ECH, dh), jnp.float32),
            pltpu.VMEM_SHARED((n_pad, dh), jnp.float32),
            pltpu.SemaphoreType.DMA,
            pltpu.SemaphoreType.DMA,
            pltpu.SemaphoreType.DMA,
            pltpu.SemaphoreType.DMA,
        ],
    )
    def agg_kernel(mlo_hbm, mhi_hbm, row_hbm, col_hbm, alo_hbm, ahi_hbm,
                   row_v, col_v, gb0, gb1, acc, gs0, gs1, ss0, ss1):
        cid = lax.axis_index("c")
        sid = lax.axis_index("s")

        @pl.loop(0, ECH)
        def _(i):
            for k in range(dh // 16):
                gb0[i, pl.ds(16 * k, 16)] = jnp.zeros((16,), jnp.float32)

        @pl.loop(0, zrows // ECH)
        def _(t):
            pltpu.sync_copy(gb0,
                            acc.at[pl.ds(sid * zrows + t * ECH, ECH)])

        plsc.subcore_barrier()

        def start_gather(j, gb, sem):
            @pl.when(cid == 0)
            def _():
                pltpu.async_copy(mlo_hbm.at[col_v.at[j]], gb, sem)

            @pl.when(cid == 1)
            def _():
                pltpu.async_copy(mhi_hbm.at[col_v.at[j]], gb, sem)

        def wait_gather(j, gb, sem):
            pltpu.make_async_copy(mlo_hbm.at[col_v.at[j]], gb, sem).wait()

        def start_scatter(j, gb, sem):
            pltpu.async_copy(gb, acc.at[row_v.at[j]], sem, add=True)

        def wait_scatter(j, gb, sem):
            pltpu.make_async_copy(gb, acc.at[row_v.at[j]], sem).wait()

        @pl.loop(0, nb)
        def _(bk):
            pltpu.sync_copy(row_hbm.at[sid * nb + bk], row_v)
            pltpu.sync_copy(col_hbm.at[sid * nb + bk], col_v)
            start_gather(0, gb0, gs0)

            @pl.loop(0, SB, step=2)
            def _(j):
                wait_gather(j, gb0, gs0)
                start_scatter(j, gb0, ss0)

                @pl.when(j > 0)
                def _():
                    wait_scatter(j - 1, gb1, ss1)

                start_gather(j + 1, gb1, gs1)
                wait_gather(j + 1, gb1, gs1)
                start_scatter(j + 1, gb1, ss1)
                wait_scatter(j, gb0, ss0)

                @pl.when(j + 2 < SB)
                def _():
                    start_gather(j + 2, gb0, gs0)

            wait_scatter(SB - 1, gb1, ss1)

        plsc.subcore_barrier()

        def _writeback(dst):
            @pl.when(sid < NS - 1)
            def _():
                pltpu.sync_copy(acc.at[pl.ds(sid * orows, orows)],
                                dst.at[pl.ds(sid * orows, orows)])

            @pl.when(sid == NS - 1)
            def _():
                pltpu.sync_copy(acc.at[pl.ds((NS - 1) * orows, last)],
                                dst.at[pl.ds((NS - 1) * orows, last)])

        @pl.when(cid == 0)
        def _():
            _writeback(alo_hbm)

        @pl.when(cid == 1)
        def _():
            _writeback(ahi_hbm)

    return agg_kernel


def _tc_linear(x, w, b2, deg2, bm, dh):
    n, d_in = x.shape
    d_out = w.shape[0]

    def body(x_ref, w_ref, b_ref, deg_ref, mlo_ref, mhi_ref):
        h = lax.dot_general(x_ref[...], w_ref[...], (((1,), (1,)), ((), ())),
                            preferred_element_type=jnp.float32,
                            precision=lax.Precision.HIGHEST)
        m = (h + b_ref[...]) * lax.rsqrt(deg_ref[...] + 1.0)
        mlo_ref[...] = m[:, :dh]
        mhi_ref[...] = m[:, dh:]

    return pl.pallas_call(
        body,
        grid=(n // bm,),
        in_specs=[
            pl.BlockSpec((bm, d_in), lambda i: (i, 0)),
            pl.BlockSpec((d_out, d_in), lambda i: (0, 0)),
            pl.BlockSpec((1, d_out), lambda i: (0, 0)),
            pl.BlockSpec((bm, 1), lambda i: (i, 0)),
        ],
        out_specs=[
            pl.BlockSpec((bm, dh), lambda i: (i, 0)),
            pl.BlockSpec((bm, dh), lambda i: (i, 0)),
        ],
        out_shape=[
            jax.ShapeDtypeStruct((n, dh), jnp.float32),
            jax.ShapeDtypeStruct((n, dh), jnp.float32),
        ],
    )(x, w, b2, deg2)


def _tc_finish(alo, ahi, mlo, mhi, deg2, bm, dh):
    n = alo.shape[0]

    def body(alo_ref, ahi_ref, mlo_ref, mhi_ref, deg_ref, out_ref):
        d = lax.rsqrt(deg_ref[...] + 1.0)
        lo = (alo_ref[...] + mlo_ref[...]) * d
        hi = (ahi_ref[...] + mhi_ref[...]) * d
        out_ref[...] = jnp.concatenate([lo, hi], axis=1)

    half = pl.BlockSpec((bm, dh), lambda i: (i, 0))
    return pl.pallas_call(
        body,
        grid=(n // bm,),
        in_specs=[half, half, half, half,
                  pl.BlockSpec((bm, 1), lambda i: (i, 0))],
        out_specs=pl.BlockSpec((bm, 2 * dh), lambda i: (i, 0)),
        out_shape=jax.ShapeDtypeStruct((n, 2 * dh), jnp.float32),
    )(alo, ahi, mlo, mhi, deg2)


def kernel(x, edge_index, W, b):
    n, d_in = x.shape
    d_out = W.shape[0]
    e = edge_index.shape[1]
    dh = d_out // 2

    e_sub = _round_up(-(-e // NS), SB * ECH)
    n_chunks = e_sub // ECH
    nb = n_chunks // SB
    e_pad = e_sub * NS
    n_pad = _round_up(n + 1, NS * ECH)

    row = edge_index[0]
    col = edge_index[1]
    pad = e_pad - e
    rowp = jnp.concatenate(
        [row, jnp.full((pad,), n, jnp.int32)]).reshape(NS * nb, SB, ECH)
    colg = jnp.concatenate(
        [col, jnp.zeros((pad,), jnp.int32)]).reshape(NS * nb, SB, ECH)
    cold = jnp.concatenate(
        [col, jnp.full((pad,), n, jnp.int32)]).reshape(NS, n_chunks, ECH)

    deg = _make_deg_kernel(n_pad, n_chunks)(cold)
    deg2 = deg[:n].reshape(n, 1)

    bm = 1000 if n % 1000 == 0 else 8
    mlo, mhi = _tc_linear(x, W, b.reshape(1, d_out), deg2, bm, dh)
    alo, ahi = _make_agg_kernel(n, n_pad, dh, n_chunks)(mlo, mhi, rowp, colg)
    return _tc_finish(alo, ahi, mlo, mhi, deg2, bm, dh)


# speedup vs baseline: 1.2968x; 1.2968x over previous
import functools

import jax
import jax.numpy as jnp
from jax import lax
from jax.experimental import pallas as pl
from jax.experimental.pallas import tpu as pltpu
from jax.experimental.pallas import tpu_sc as plsc

NC = 2
NS = 16
ECH = 128
SB = 40


def _round_up(a, b):
    return (a + b - 1) // b * b


def _sc_mesh():
    return plsc.VectorSubcoreMesh(
        core_axis_name="c", subcore_axis_name="s", num_cores=NC,
        num_subcores=NS)


def _make_deg_kernel(n_pad, n_chunks):
    zrows = n_pad // NS

    @functools.partial(
        pl.kernel,
        out_type=jax.ShapeDtypeStruct((n_pad,), jnp.float32),
        mesh=_sc_mesh(),
        scratch_types=[
            pltpu.VMEM((n_chunks, ECH), jnp.int32),
            pltpu.VMEM((ECH,), jnp.float32),
            pltpu.VMEM((zrows,), jnp.float32),
            pltpu.VMEM_SHARED((n_pad,), jnp.float32),
        ],
    )
    def deg_kernel(col_hbm, deg_hbm, idx_v, ones_v, zero_v, acc):
        cid = lax.axis_index("c")
        sid = lax.axis_index("s")

        @pl.when(cid == 0)
        def _():
            for k in range(ECH // 16):
                ones_v[pl.ds(16 * k, 16)] = jnp.ones((16,), jnp.float32)

            @pl.loop(0, zrows // 16)
            def _(i):
                zero_v[pl.ds(i * 16, 16)] = jnp.zeros((16,), jnp.float32)

            pltpu.sync_copy(zero_v, acc.at[pl.ds(sid * zrows, zrows)])
            pltpu.sync_copy(col_hbm.at[sid], idx_v)
            plsc.subcore_barrier()

            @pl.loop(0, n_chunks)
            def _(j):
                pltpu.sync_copy(ones_v, acc.at[idx_v.at[j]], add=True)

            plsc.subcore_barrier()
            pltpu.sync_copy(acc.at[pl.ds(sid * zrows, zrows)],
                            deg_hbm.at[pl.ds(sid * zrows, zrows)])

    return deg_kernel


def _make_agg_kernel(n, n_pad, dh, n_chunks):
    orows = _round_up(-(-n // NS), 8)
    last = n - orows * (NS - 1)

    @functools.partial(
        pl.kernel,
        out_type=(jax.ShapeDtypeStruct((n, dh), jnp.float32),
                  jax.ShapeDtypeStruct((n, dh), jnp.float32)),
        mesh=_sc_mesh(),
        scratch_types=[
            pltpu.VMEM((n_chunks, ECH), jnp.int32),
            pltpu.VMEM((n_chunks, ECH), jnp.int32),
            pltpu.VMEM((ECH, dh), jnp.float32),
            pltpu.VMEM_SHARED((n_pad, dh), jnp.float32),
        ],
    )
    def agg_kernel(mlo_hbm, mhi_hbm, row_hbm, col_hbm, alo_hbm, ahi_hbm,
                   row_v, col_v, gbuf, acc):
        cid = lax.axis_index("c")
        sid = lax.axis_index("s")

        def _stage(src):
            @pl.when(sid < NS - 1)
            def _():
                pltpu.sync_copy(src.at[pl.ds(sid * orows, orows)],
                                acc.at[pl.ds(sid * orows, orows)])

            @pl.when(sid == NS - 1)
            def _():
                pltpu.sync_copy(src.at[pl.ds((NS - 1) * orows, last)],
                                acc.at[pl.ds((NS - 1) * orows, last)])

        @pl.when(cid == 0)
        def _():
            _stage(mlo_hbm)

        @pl.when(cid == 1)
        def _():
            _stage(mhi_hbm)

        pltpu.sync_copy(row_hbm.at[sid], row_v)
        pltpu.sync_copy(col_hbm.at[sid], col_v)
        plsc.subcore_barrier()

        @pl.loop(0, n_chunks)
        def _(j):
            @pl.when(cid == 0)
            def _():
                pltpu.sync_copy(mlo_hbm.at[col_v.at[j]], gbuf)

            @pl.when(cid == 1)
            def _():
                pltpu.sync_copy(mhi_hbm.at[col_v.at[j]], gbuf)

            pltpu.sync_copy(gbuf, acc.at[row_v.at[j]], add=True)

        plsc.subcore_barrier()

        def _writeback(dst):
            @pl.when(sid < NS - 1)
            def _():
                pltpu.sync_copy(acc.at[pl.ds(sid * orows, orows)],
                                dst.at[pl.ds(sid * orows, orows)])

            @pl.when(sid == NS - 1)
            def _():
                pltpu.sync_copy(acc.at[pl.ds((NS - 1) * orows, last)],
                                dst.at[pl.ds((NS - 1) * orows, last)])

        @pl.when(cid == 0)
        def _():
            _writeback(alo_hbm)

        @pl.when(cid == 1)
        def _():
            _writeback(ahi_hbm)

    return agg_kernel


def _tc_linear(x, w, b2, deg2, bm, dh):
    n, d_in = x.shape
    d_out = w.shape[0]

    def body(x_ref, w_ref, b_ref, deg_ref, mlo_ref, mhi_ref):
        h = lax.dot_general(x_ref[...], w_ref[...], (((1,), (1,)), ((), ())),
                            preferred_element_type=jnp.float32)
        m = (h + b_ref[...]) * lax.rsqrt(deg_ref[...] + 1.0)
        mlo_ref[...] = m[:, :dh]
        mhi_ref[...] = m[:, dh:]

    return pl.pallas_call(
        body,
        grid=(n // bm,),
        in_specs=[
            pl.BlockSpec((bm, d_in), lambda i: (i, 0)),
            pl.BlockSpec((d_out, d_in), lambda i: (0, 0)),
            pl.BlockSpec((1, d_out), lambda i: (0, 0)),
            pl.BlockSpec((bm, 1), lambda i: (i, 0)),
        ],
        out_specs=[
            pl.BlockSpec((bm, dh), lambda i: (i, 0)),
            pl.BlockSpec((bm, dh), lambda i: (i, 0)),
        ],
        out_shape=[
            jax.ShapeDtypeStruct((n, dh), jnp.float32),
            jax.ShapeDtypeStruct((n, dh), jnp.float32),
        ],
    )(x, w, b2, deg2)


def _tc_finish(alo, ahi, deg2, bm, dh):
    n = alo.shape[0]

    def body(alo_ref, ahi_ref, deg_ref, out_ref):
        d = lax.rsqrt(deg_ref[...] + 1.0)
        out_ref[...] = jnp.concatenate(
            [alo_ref[...] * d, ahi_ref[...] * d], axis=1)

    half = pl.BlockSpec((bm, dh), lambda i: (i, 0))
    return pl.pallas_call(
        body,
        grid=(n // bm,),
        in_specs=[half, half,
                  pl.BlockSpec((bm, 1), lambda i: (i, 0))],
        out_specs=pl.BlockSpec((bm, 2 * dh), lambda i: (i, 0)),
        out_shape=jax.ShapeDtypeStruct((n, 2 * dh), jnp.float32),
    )(alo, ahi, deg2)


def kernel(x, edge_index, W, b):
    n, d_in = x.shape
    d_out = W.shape[0]
    e = edge_index.shape[1]
    dh = d_out // 2

    e_sub = _round_up(-(-e // NS), ECH)
    n_chunks = e_sub // ECH
    e_pad = e_sub * NS
    n_pad = _round_up(n + 1, NS * ECH)

    row = edge_index[0]
    col = edge_index[1]
    pad = e_pad - e
    rowp = jnp.concatenate(
        [row, jnp.full((pad,), n, jnp.int32)]).reshape(NS, n_chunks, ECH)
    colg = jnp.concatenate(
        [col, jnp.zeros((pad,), jnp.int32)]).reshape(NS, n_chunks, ECH)
    cold = jnp.concatenate(
        [col, jnp.full((pad,), n, jnp.int32)]).reshape(NS, n_chunks, ECH)

    deg = _make_deg_kernel(n_pad, n_chunks)(cold)
    deg2 = deg[:n].reshape(n, 1)

    bm = 1000 if n % 1000 == 0 else 8
    mlo, mhi = _tc_linear(x, W, b.reshape(1, d_out), deg2, bm, dh)
    alo, ahi = _make_agg_kernel(n, n_pad, dh, n_chunks)(mlo, mhi, rowp, colg)
    return _tc_finish(alo, ahi, deg2, bm, dh)
